# baseline (speedup 1.0000x reference)
import jax
import jax.numpy as jnp
from jax import lax
from jax.experimental import pallas as pl
from jax.experimental.pallas import tpu as pltpu
from jax.experimental.pallas import tpu_sc as plsc

EMBED = 64
NC = 2
NS = 16
NW = NC * NS
L = 16
CHUNK = 128
SC_FRAC = 2


def _sc_body(sg_idx_hbm, wy_idx_hbm, mu_idx_hbm, sg_hbm, wy_hbm, mu_hbm,
             out_hbm, sgi_v, wyi_v, mui_v, wy_v, mu_v, out_v,
             sem, sem_idx, pre_sems):
    bpw = out_v.shape[0]
    nch = sgi_v.shape[0]
    wid = lax.axis_index("s") * NC + lax.axis_index("c")
    base = wid * bpw
    idx_cp = pltpu.async_copy(
        sg_idx_hbm.at[pl.ds(wid * nch, nch), :], sgi_v, sem_idx)
    tab_cps = [
        pltpu.async_copy(wy_idx_hbm.at[pl.ds(base, bpw)], wyi_v, sem),
        pltpu.async_copy(mu_idx_hbm.at[pl.ds(base, bpw)], mui_v, sem),
        pltpu.async_copy(wy_hbm, wy_v, sem),
        pltpu.async_copy(mu_hbm, mu_v, sem),
    ]
    idx_cp.wait()
    pre_cps = [
        pltpu.async_copy(sg_hbm.at[sgi_v.at[j]],
                         out_v.at[pl.ds(j * CHUNK, CHUNK), :], pre_sems[j])
        for j in range(nch)
    ]
    for cp in tab_cps:
        cp.wait()
    for cp in pre_cps:
        cp.wait()

    lanes = lax.iota(jnp.int32, L)
    dv = [(lanes + r) & (L - 1) for r in range(L)]

    def group(g, carry):
        off = g * L
        wyi = wyi_v[pl.ds(off, L)] * EMBED
        mui = mui_v[pl.ds(off, L)] * EMBED
        rows = lanes + off
        for q in range(EMBED // L):
            wq = wyi + q * L
            mq = mui + q * L
            for r in range(L):
                val = (plsc.load_gather(wy_v, [wq + dv[r]])
                       + plsc.load_gather(mu_v, [mq + dv[r]]))
                plsc.addupdate_scatter(out_v, [rows, dv[r] + q * L], val)
        return carry

    lax.fori_loop(0, bpw // L, group, 0)
    pltpu.sync_copy(out_v, out_hbm.at[pl.ds(base, bpw), :])


def _tc_body(sgi_ref, wyi_ref, mui_ref, sgt_ref, wyt_ref, mut_ref, out_ref):
    def lookup(idx_col, table_ref):
        n = table_ref.shape[0]
        onehot = (idx_col == lax.broadcasted_iota(jnp.int32, (1, n), 1)
                  ).astype(jnp.float32)
        return jnp.dot(onehot, table_ref[...],
                       preferred_element_type=jnp.float32)
    out_ref[...] = (lookup(sgi_ref[...], sgt_ref)
                    + lookup(wyi_ref[...], wyt_ref)
                    + lookup(mui_ref[...], mut_ref))


def kernel(space_group, wyckoff_letter, multiplicity, sg_table,
           wyckoff_table, mult_table):
    B = space_group.shape[0]
    P = B // SC_FRAC
    bpw = P // NW
    nch = bpw // CHUNK
    sg = space_group.astype(jnp.int32)
    wy = wyckoff_letter.astype(jnp.int32)
    mu = multiplicity.astype(jnp.int32)

    mesh = plsc.VectorSubcoreMesh(core_axis_name="c", subcore_axis_name="s")
    sc_run = pl.kernel(
        _sc_body,
        mesh=mesh,
        compiler_params=pltpu.CompilerParams(needs_layout_passes=False,
                                             use_tc_tiling_on_sc=False),
        out_type=jax.ShapeDtypeStruct((P, EMBED), jnp.float32),
        scratch_types=[
            pltpu.VMEM((nch, CHUNK), jnp.int32),
            pltpu.VMEM((bpw,), jnp.int32),
            pltpu.VMEM((bpw,), jnp.int32),
            pltpu.VMEM((wyckoff_table.size,), jnp.float32),
            pltpu.VMEM((mult_table.size,), jnp.float32),
            pltpu.VMEM((bpw, EMBED), jnp.float32),
            pltpu.SemaphoreType.DMA,
            pltpu.SemaphoreType.DMA,
            [pltpu.SemaphoreType.DMA for _ in range(nch)],
        ],
    )
    sc_out = sc_run(sg[:P].reshape(P // CHUNK, CHUNK), wy, mu, sg_table,
                    wyckoff_table.reshape(-1), mult_table.reshape(-1))

    tc_run = pl.pallas_call(
        _tc_body,
        out_shape=jax.ShapeDtypeStruct((B - P, EMBED), jnp.float32),
    )
    tc_out = tc_run(sg[P:, None], wy[P:, None], mu[P:, None],
                    sg_table, wyckoff_table, mult_table)

    out = jnp.empty((B, EMBED), jnp.float32)
    out = lax.dynamic_update_slice(out, sc_out, (0, 0))
    return lax.dynamic_update_slice(out, tc_out, (P, 0))


# speedup vs baseline: 1.0538x; 1.0538x over previous
import jax
import jax.numpy as jnp
from jax import lax
from jax.experimental import pallas as pl
from jax.experimental.pallas import tpu as pltpu
from jax.experimental.pallas import tpu_sc as plsc

EMBED = 64
NC = 2
NS = 16
NW = NC * NS
L = 16


def _body(sg_idx_hbm, wy_idx_hbm, mu_idx_hbm, sg_hbm, wy_hbm, mu_hbm,
          out_hbm, sgi_v, wyi_v, mui_v, sg_v, wy_v, mu_v, out_v, sem):
    bpw = sgi_v.shape[0]
    wid = lax.axis_index("s") * NC + lax.axis_index("c")
    base = wid * bpw
    cps = [
        pltpu.async_copy(sg_idx_hbm.at[pl.ds(base, bpw)], sgi_v, sem),
        pltpu.async_copy(wy_idx_hbm.at[pl.ds(base, bpw)], wyi_v, sem),
        pltpu.async_copy(mu_idx_hbm.at[pl.ds(base, bpw)], mui_v, sem),
        pltpu.async_copy(sg_hbm, sg_v, sem),
        pltpu.async_copy(wy_hbm, wy_v, sem),
        pltpu.async_copy(mu_hbm, mu_v, sem),
    ]
    for cp in cps:
        cp.wait()

    lanes = lax.iota(jnp.int32, L)

    def group(g, carry):
        off = g * L
        sgi = sgi_v[pl.ds(off, L)] * EMBED
        wyi = wyi_v[pl.ds(off, L)] * EMBED
        mui = mui_v[pl.ds(off, L)] * EMBED
        row = (lanes + off) * EMBED
        for d in range(EMBED):
            dvec = (lanes + d) & (EMBED - 1)
            r = (plsc.load_gather(sg_v, [sgi + dvec])
                 + plsc.load_gather(wy_v, [wyi + dvec])
                 + plsc.load_gather(mu_v, [mui + dvec]))
            plsc.store_scatter(out_v, [row + dvec], r)
        return carry

    lax.fori_loop(0, bpw // L, group, 0)
    pltpu.sync_copy(out_v, out_hbm.at[pl.ds(base * EMBED, bpw * EMBED)])


def kernel(space_group, wyckoff_letter, multiplicity, sg_table,
           wyckoff_table, mult_table):
    B = space_group.shape[0]
    bpw = B // NW
    sg = space_group.astype(jnp.int32)
    wy = wyckoff_letter.astype(jnp.int32)
    mu = multiplicity.astype(jnp.int32)
    mesh = plsc.VectorSubcoreMesh(core_axis_name="c", subcore_axis_name="s")
    run = pl.kernel(
        _body,
        mesh=mesh,
        compiler_params=pltpu.CompilerParams(needs_layout_passes=False),
        out_type=jax.ShapeDtypeStruct((B * EMBED,), jnp.float32),
        scratch_types=[
            pltpu.VMEM((bpw,), jnp.int32),
            pltpu.VMEM((bpw,), jnp.int32),
            pltpu.VMEM((bpw,), jnp.int32),
            pltpu.VMEM((sg_table.size,), jnp.float32),
            pltpu.VMEM((wyckoff_table.size,), jnp.float32),
            pltpu.VMEM((mult_table.size,), jnp.float32),
            pltpu.VMEM((bpw * EMBED,), jnp.float32),
            pltpu.SemaphoreType.DMA,
        ],
    )
    out = run(sg, wy, mu, sg_table.reshape(-1), wyckoff_table.reshape(-1),
              mult_table.reshape(-1))
    return out.reshape(B, EMBED)
